# Initial kernel scaffold; baseline (speedup 1.0000x reference)
import jax
import jax.numpy as jnp
from jax.experimental import pallas as pl


def kernel(x, edge_index, Wl1, bl1, Wr1, ln1_w, ln1_b, Wl2, bl2, Wr2, ln2_w, ln2_b):
    raise NotImplementedError("write your pallas kernel here")


# speedup vs baseline: 2.7392x; 2.7392x over previous
import functools

import jax
import jax.numpy as jnp
from jax import lax
from jax.experimental import pallas as pl
from jax.experimental.pallas import tpu as pltpu
from jax.experimental.pallas import tpu_sc as plsc

N = 10000
D = 128
E = 320000

NW = 32
NP = 10240
ECHUNK = 64
EROWS = E // ECHUNK
EROWS_PAD = 5120
RPW = EROWS_PAD // NW
PAD_DST = NP - 8
ROWS_PER_SUB = NP // 16

NB = 1000
GRID = N // NB
CW = 128


_MESH = plsc.VectorSubcoreMesh(core_axis_name="c", subcore_axis_name="s")


def _sc_scatter_body(table, srcr, dstr, ssum_out, src_v, dst_v, rows_v,
                     acc_sh, sem):
  c = lax.axis_index("c")
  s = lax.axis_index("s")
  wid = s * 2 + c

  pltpu.sync_copy(srcr.at[pl.ds(wid * RPW, RPW)], src_v)
  pltpu.sync_copy(dstr.at[pl.ds(wid * RPW, RPW)], dst_v)

  zero16 = jnp.zeros((16,), jnp.float32)

  def zbody(i, carry):
    rows_v[i // 8, pl.ds((i % 8) * 16, 16)] = zero16
    return carry

  lax.fori_loop(0, ECHUNK * 8, zbody, 0)

  def zcpy(k, carry):
    pltpu.sync_copy(
        rows_v, acc_sh.at[pl.ds(s * ROWS_PER_SUB + k * ECHUNK, ECHUNK)])
    return carry

  lax.fori_loop(0, ROWS_PER_SUB // ECHUNK, zcpy, 0)

  plsc.subcore_barrier()

  def ebody(j, carry):
    pltpu.async_copy(table.at[src_v.at[j]], rows_v, sem).wait()
    pltpu.sync_copy(rows_v, acc_sh.at[dst_v.at[j]], add=True)
    return carry

  lax.fori_loop(0, RPW, ebody, 0)

  plsc.subcore_barrier()

  def obody(k, carry):
    r0 = s * ROWS_PER_SUB + k * 128
    pltpu.sync_copy(acc_sh.at[pl.ds(r0, 128)], ssum_out.at[c, pl.ds(r0, 128)])
    return carry

  lax.fori_loop(0, ROWS_PER_SUB // 128, obody, 0)


_sc_scatter = functools.partial(
    pl.kernel, mesh=_MESH,
    out_type=[jax.ShapeDtypeStruct((2, NP, D), jnp.float32)],
    scratch_types=[
        pltpu.VMEM((RPW, ECHUNK), jnp.int32),
        pltpu.VMEM((RPW, ECHUNK), jnp.int32),
        pltpu.VMEM((ECHUNK, D), jnp.float32),
        pltpu.VMEM_SHARED((NP, D), jnp.float32),
        pltpu.SemaphoreType.DMA,
    ])(_sc_scatter_body)


def _sc_count_body(dstr, cnt_out, dst_v, ones_v, zc_v, cnt_sh):
  c = lax.axis_index("c")
  s = lax.axis_index("s")
  wid = s * 2 + c

  pltpu.sync_copy(dstr.at[pl.ds(wid * RPW, RPW)], dst_v)

  zero16 = jnp.zeros((16,), jnp.float32)
  ones16 = jnp.ones((16,), jnp.float32)

  def cinit(i, carry):
    ones_v[i // 8, pl.ds((i % 8) * 16, 16)] = ones16
    zc_v[i // 8, pl.ds((i % 8) * 16, 16)] = zero16
    return carry

  lax.fori_loop(0, ECHUNK * 8, cinit, 0)

  def ccpy(k, carry):
    pltpu.sync_copy(
        zc_v, cnt_sh.at[pl.ds(s * ROWS_PER_SUB + k * ECHUNK, ECHUNK)])
    return carry

  lax.fori_loop(0, ROWS_PER_SUB // ECHUNK, ccpy, 0)

  plsc.subcore_barrier()

  def ebody(j, carry):
    pltpu.sync_copy(ones_v, cnt_sh.at[dst_v.at[j]], add=True)
    return carry

  lax.fori_loop(0, RPW, ebody, 0)

  plsc.subcore_barrier()

  def obody(k, carry):
    r0 = s * ROWS_PER_SUB + k * 128
    pltpu.sync_copy(cnt_sh.at[pl.ds(r0, 128)], cnt_out.at[c, pl.ds(r0, 128)])
    return carry

  lax.fori_loop(0, ROWS_PER_SUB // 128, obody, 0)


_sc_count = functools.partial(
    pl.kernel, mesh=_MESH,
    out_type=[jax.ShapeDtypeStruct((2, NP, CW), jnp.float32)],
    scratch_types=[
        pltpu.VMEM((RPW, ECHUNK), jnp.int32),
        pltpu.VMEM((ECHUNK, CW), jnp.float32),
        pltpu.VMEM((ECHUNK, CW), jnp.float32),
        pltpu.VMEM_SHARED((NP, CW), jnp.float32),
    ])(_sc_count_body)


def _lin_body(sp, cp, xr, wl, wr, b, z_out, st_out, acc):
  i = pl.program_id(0)
  parts = sp[...]
  ssum = parts[0] + parts[1]
  cnt = cp[0, :, 0:1] + cp[1, :, 0:1]
  inv = 1.0 / jnp.maximum(cnt, 1.0)
  agg = ssum * inv
  z = (jnp.dot(agg, wl[...], preferred_element_type=jnp.float32)
       + jnp.dot(xr[...], wr[...], preferred_element_type=jnp.float32)
       + b[...])
  z_out[...] = z
  ps = jnp.sum(z)
  pq = jnp.sum(z * z)

  @pl.when(i == 0)
  def _():
    acc[0] = ps
    acc[1] = pq

  @pl.when(i > 0)
  def _():
    acc[0] += ps
    acc[1] += pq

  @pl.when(i == pl.num_programs(0) - 1)
  def _():
    row = lax.broadcasted_iota(jnp.int32, (8, 128), 0)
    st_out[...] = jnp.where(row == 0, acc[0],
                            jnp.where(row == 1, acc[1], 0.0))


def _lin_call(ssum_parts, cnt_t, xr, wl_t, wr_t, b2d):
  return pl.pallas_call(
      _lin_body,
      grid=(GRID,),
      in_specs=[
          pl.BlockSpec((2, NB, D), lambda i: (0, i, 0)),
          pl.BlockSpec((2, NB, CW), lambda i: (0, i, 0)),
          pl.BlockSpec((NB, D), lambda i: (i, 0)),
          pl.BlockSpec((D, D), lambda i: (0, 0)),
          pl.BlockSpec((D, D), lambda i: (0, 0)),
          pl.BlockSpec((1, D), lambda i: (0, 0)),
      ],
      out_specs=[
          pl.BlockSpec((NB, D), lambda i: (i, 0)),
          pl.BlockSpec((8, 128), lambda i: (0, 0)),
      ],
      out_shape=[
          jax.ShapeDtypeStruct((N, D), jnp.float32),
          jax.ShapeDtypeStruct((8, 128), jnp.float32),
      ],
      scratch_shapes=[pltpu.SMEM((2,), jnp.float32)],
  )(ssum_parts, cnt_t, xr, wl_t, wr_t, b2d)


_INV_ND = 1.0 / (N * D)


def _ln_body(z, st, w, b, o):
  s1 = jnp.sum(st[0:1, 0:1])
  s2 = jnp.sum(st[1:2, 0:1])
  m = s1 * _INV_ND
  var = s2 * _INV_ND - m * m
  std = jnp.sqrt(jnp.maximum(var, 0.0))
  rd = 1.0 / (std + 1e-5)
  o[...] = jnp.maximum((z[...] - m) * rd * w[...] + b[...], 0.0)


def _ln_res_body(z, st, w, b, res, o):
  s1 = jnp.sum(st[0:1, 0:1])
  s2 = jnp.sum(st[1:2, 0:1])
  m = s1 * _INV_ND
  var = s2 * _INV_ND - m * m
  std = jnp.sqrt(jnp.maximum(var, 0.0))
  rd = 1.0 / (std + 1e-5)
  o[...] = jnp.maximum((z[...] - m) * rd * w[...] + b[...], 0.0) + res[...]


def _ln_call(z, st, w2d, b2d, res=None):
  blk = pl.BlockSpec((NB, D), lambda i: (i, 0))
  in_specs = [
      blk,
      pl.BlockSpec((8, 128), lambda i: (0, 0)),
      pl.BlockSpec((1, D), lambda i: (0, 0)),
      pl.BlockSpec((1, D), lambda i: (0, 0)),
  ]
  args = [z, st, w2d, b2d]
  body = _ln_body
  if res is not None:
    in_specs.append(blk)
    args.append(res)
    body = _ln_res_body
  return pl.pallas_call(
      body,
      grid=(GRID,),
      in_specs=in_specs,
      out_specs=blk,
      out_shape=jax.ShapeDtypeStruct((N, D), jnp.float32),
  )(*args)


def kernel(x, edge_index, Wl1, bl1, Wr1, ln1_w, ln1_b, Wl2, bl2, Wr2, ln2_w, ln2_b):
  src = edge_index[0]
  dst = edge_index[1]
  pad = EROWS_PAD * ECHUNK - E
  src_p = jnp.concatenate(
      [src, jnp.zeros((pad,), jnp.int32)]).reshape(EROWS_PAD, ECHUNK)
  dst_p = jnp.concatenate(
      [dst, jnp.full((pad,), PAD_DST, jnp.int32)]).reshape(EROWS_PAD, ECHUNK)

  wl1_t = Wl1.T
  wr1_t = Wr1.T
  wl2_t = Wl2.T
  wr2_t = Wr2.T
  bl1_2 = bl1.reshape(1, D)
  bl2_2 = bl2.reshape(1, D)
  ln1w = ln1_w.reshape(1, D)
  ln1b = ln1_b.reshape(1, D)
  ln2w = ln2_w.reshape(1, D)
  ln2b = ln2_b.reshape(1, D)

  (cnt_parts,) = _sc_count(dst_p)
  (ssum1,) = _sc_scatter(x, src_p, dst_p)

  z1, st1 = _lin_call(ssum1, cnt_parts, x, wl1_t, wr1_t, bl1_2)
  h1 = _ln_call(z1, st1, ln1w, ln1b)

  (ssum2,) = _sc_scatter(h1, src_p, dst_p)
  z2, st2 = _lin_call(ssum2, cnt_parts, h1, wl2_t, wr2_t, bl2_2)
  out = _ln_call(z2, st2, ln2w, ln2b, res=x)

  return (out, edge_index)
